# Initial kernel scaffold; baseline (speedup 1.0000x reference)
import jax
import jax.numpy as jnp
from jax.experimental import pallas as pl


def kernel(x, edge_index, edge_attr, W_msg, b_msg, W_un, b_un, W_ue, b_ue):
    raise NotImplementedError("write your pallas kernel here")


# speedup vs baseline: 3.0449x; 3.0449x over previous
import functools

import jax
import jax.numpy as jnp
from jax import lax
from jax.experimental import pallas as pl
from jax.experimental.pallas import tpu as pltpu
from jax.experimental.pallas import tpu_sc as plsc

NC = 2
NS = 16
L = 16
K = 128


def _sc_tanh(z):
    a = jnp.abs(z)
    e = jnp.exp(a * -2.0)
    return jnp.sign(z) * ((1.0 - e) / (1.0 + e))


def _tc_node_pre(x, Wm_x, Wue_s, b_msg, *, blk):
    n, d = x.shape
    grid = n // blk

    def body(x_ref, wmx_ref, wues_ref, bm_ref, px_ref, qa_ref):
        xb = x_ref[...]
        px_ref[...] = (
            jnp.dot(xb, wmx_ref[...], preferred_element_type=jnp.float32)
            + bm_ref[...][None, :]
        )
        qa_ref[...] = jnp.dot(xb, wues_ref[...], preferred_element_type=jnp.float32)

    return pl.pallas_call(
        body,
        grid=(grid,),
        in_specs=[
            pl.BlockSpec((blk, d), lambda i: (i, 0)),
            pl.BlockSpec(Wm_x.shape, lambda i: (0, 0)),
            pl.BlockSpec(Wue_s.shape, lambda i: (0, 0)),
            pl.BlockSpec(b_msg.shape, lambda i: (0,)),
        ],
        out_specs=[
            pl.BlockSpec((blk, Wm_x.shape[1]), lambda i: (i, 0)),
            pl.BlockSpec((blk, Wue_s.shape[1]), lambda i: (i, 0)),
        ],
        out_shape=[
            jax.ShapeDtypeStruct((n, Wm_x.shape[1]), jnp.float32),
            jax.ShapeDtypeStruct((n, Wue_s.shape[1]), jnp.float32),
        ],
    )(x, Wm_x, Wue_s, b_msg)


def _tc_edge_pre(ea, Wm_e, Wue_e, b_ue, *, blk):
    e, de = ea.shape
    grid = e // blk

    def body(ea_ref, wme_ref, wuee_ref, bue_ref, pe_ref, pe2_ref):
        eb = ea_ref[...]
        pe_ref[...] = jnp.dot(eb, wme_ref[...], preferred_element_type=jnp.float32)
        pe2_ref[...] = (
            jnp.dot(eb, wuee_ref[...], preferred_element_type=jnp.float32)
            + bue_ref[...][None, :]
        )

    return pl.pallas_call(
        body,
        grid=(grid,),
        in_specs=[
            pl.BlockSpec((blk, de), lambda i: (i, 0)),
            pl.BlockSpec(Wm_e.shape, lambda i: (0, 0)),
            pl.BlockSpec(Wue_e.shape, lambda i: (0, 0)),
            pl.BlockSpec(b_ue.shape, lambda i: (0,)),
        ],
        out_specs=[
            pl.BlockSpec((blk, Wm_e.shape[1]), lambda i: (i, 0)),
            pl.BlockSpec((blk, Wue_e.shape[1]), lambda i: (i, 0)),
        ],
        out_shape=[
            jax.ShapeDtypeStruct((e, Wm_e.shape[1]), jnp.float32),
            jax.ShapeDtypeStruct((e, Wue_e.shape[1]), jnp.float32),
        ],
    )(ea, Wm_e, Wue_e, b_ue)


def _sc_aggregate(px, pe, src, dst):
    n, d = px.shape
    e = src.shape[0]
    nblk = e // K
    zc = 80
    nzc = n // zc
    mesh = plsc.VectorSubcoreMesh(core_axis_name="c", subcore_axis_name="s")

    @functools.partial(
        pl.kernel,
        out_type=[
            jax.ShapeDtypeStruct((NC, n, d), jnp.float32),
            jax.ShapeDtypeStruct((NC * NS * n,), jnp.float32),
        ],
        mesh=mesh,
        scratch_types=[
            pltpu.VMEM((K,), jnp.int32),
            pltpu.VMEM((K,), jnp.int32),
            pltpu.VMEM((K, d), jnp.float32),
            pltpu.VMEM((K, d), jnp.float32),
            pltpu.VMEM((n,), jnp.float32),
            pltpu.VMEM_SHARED((n, d), jnp.float32),
            pltpu.SemaphoreType.DMA,
            pltpu.SemaphoreType.DMA,
            pltpu.SemaphoreType.DMA,
            pltpu.SemaphoreType.DMA,
        ],
        compiler_params=pltpu.CompilerParams(
            needs_layout_passes=False, use_tc_tiling_on_sc=False
        ),
    )
    def run(px_hbm, pe_hbm, src_hbm, dst_hbm, out_hbm, cnt_hbm,
            sidx, didx, rows, pev, cnt, acc, s_si, s_di, s_pe, s_g):
        cid = lax.axis_index("c")
        sid = lax.axis_index("s")
        wid = sid * NC + cid

        zeros = jnp.zeros((L,), jnp.float32)
        ones = jnp.ones((L,), jnp.float32)

        @pl.loop(0, n // L)
        def _(i):
            cnt[pl.ds(i * L, L)] = zeros

        @pl.loop(0, K)
        def _(r):
            for c in range(d // L):
                rows[r, pl.ds(c * L, L)] = zeros

        @pl.loop(sid, nzc, step=NS)
        def _(j):
            pltpu.sync_copy(rows.at[pl.ds(0, zc)], acc.at[pl.ds(j * zc, zc)])

        plsc.subcore_barrier()

        @pl.loop(wid, nblk, step=NC * NS)
        def _(b):
            base = b * K
            a_si = pltpu.async_copy(src_hbm.at[pl.ds(base, K)], sidx, s_si)
            a_di = pltpu.async_copy(dst_hbm.at[pl.ds(base, K)], didx, s_di)
            a_pe = pltpu.async_copy(pe_hbm.at[pl.ds(base, K), :], pev, s_pe)
            a_si.wait()
            a_g = pltpu.async_copy(px_hbm.at[sidx], rows, s_g)
            a_di.wait()

            for c in range(K // L):
                idxv = didx[pl.ds(c * L, L)]
                plsc.addupdate_scatter(cnt, [idxv], ones)

            a_pe.wait()
            a_g.wait()

            @pl.loop(0, K)
            def _(r):
                for c in range(d // L):
                    z = rows[r, pl.ds(c * L, L)] + pev[r, pl.ds(c * L, L)]
                    rows[r, pl.ds(c * L, L)] = _sc_tanh(z)

            pltpu.sync_copy(rows, acc.at[didx], add=True)

        plsc.subcore_barrier()

        @pl.loop(sid, nzc, step=NS)
        def _(j):
            r0 = j * zc
            pltpu.sync_copy(acc.at[pl.ds(r0, zc)], out_hbm.at[cid, pl.ds(r0, zc), :])

        pltpu.sync_copy(cnt, cnt_hbm.at[pl.ds(wid * n, n)])

    return run(px, pe, src, dst)


def _tc_node_update(acc, cnts, x, Wu_n, Wu_x, b_un, Wue_d, *, blk):
    n, d = x.shape
    grid = n // blk
    nw = cnts.shape[1]

    def body(acc_ref, cnt_ref, x_ref, wun_ref, wux_ref, bun_ref, wued_ref,
             h_ref, qb_ref):
        a = acc_ref[0] + acc_ref[1]
        cnt = jnp.maximum(jnp.sum(cnt_ref[...], axis=1), 1.0)[:, None]
        neigh = a / cnt
        h = jnp.tanh(
            jnp.dot(neigh, wun_ref[...], preferred_element_type=jnp.float32)
            + jnp.dot(x_ref[...], wux_ref[...], preferred_element_type=jnp.float32)
            + bun_ref[...][None, :]
        )
        h_ref[...] = h
        qb_ref[...] = jnp.dot(h, wued_ref[...], preferred_element_type=jnp.float32)

    return pl.pallas_call(
        body,
        grid=(grid,),
        in_specs=[
            pl.BlockSpec((NC, blk, d), lambda i: (0, i, 0)),
            pl.BlockSpec((blk, nw), lambda i: (i, 0)),
            pl.BlockSpec((blk, d), lambda i: (i, 0)),
            pl.BlockSpec(Wu_n.shape, lambda i: (0, 0)),
            pl.BlockSpec(Wu_x.shape, lambda i: (0, 0)),
            pl.BlockSpec(b_un.shape, lambda i: (0,)),
            pl.BlockSpec(Wue_d.shape, lambda i: (0, 0)),
        ],
        out_specs=[
            pl.BlockSpec((blk, Wu_n.shape[1]), lambda i: (i, 0)),
            pl.BlockSpec((blk, Wue_d.shape[1]), lambda i: (i, 0)),
        ],
        out_shape=[
            jax.ShapeDtypeStruct((n, Wu_n.shape[1]), jnp.float32),
            jax.ShapeDtypeStruct((n, Wue_d.shape[1]), jnp.float32),
        ],
    )(acc, cnts, x, Wu_n, Wu_x, b_un, Wue_d)


def _sc_edge_out(qa, qb, pe2, src, dst):
    n, de = qa.shape
    e = src.shape[0]
    nblk = e // K
    mesh = plsc.VectorSubcoreMesh(core_axis_name="c", subcore_axis_name="s")

    @functools.partial(
        pl.kernel,
        out_type=jax.ShapeDtypeStruct((e, de), jnp.float32),
        mesh=mesh,
        scratch_types=[
            pltpu.VMEM((K,), jnp.int32),
            pltpu.VMEM((K,), jnp.int32),
            pltpu.VMEM((K, de), jnp.float32),
            pltpu.VMEM((K, de), jnp.float32),
            pltpu.VMEM((K, de), jnp.float32),
            pltpu.VMEM((K, de), jnp.float32),
            pltpu.SemaphoreType.DMA,
            pltpu.SemaphoreType.DMA,
            pltpu.SemaphoreType.DMA,
            pltpu.SemaphoreType.DMA,
            pltpu.SemaphoreType.DMA,
        ],
        compiler_params=pltpu.CompilerParams(
            needs_layout_passes=False, use_tc_tiling_on_sc=False
        ),
    )
    def run(qa_hbm, qb_hbm, pe2_hbm, src_hbm, dst_hbm, out_hbm,
            sidx, didx, qav, qbv, pev, ev, s_si, s_di, s_pe, s_ga, s_gb):
        cid = lax.axis_index("c")
        sid = lax.axis_index("s")
        wid = sid * NC + cid

        @pl.loop(wid, nblk, step=NC * NS)
        def _(b):
            base = b * K
            a_si = pltpu.async_copy(src_hbm.at[pl.ds(base, K)], sidx, s_si)
            a_di = pltpu.async_copy(dst_hbm.at[pl.ds(base, K)], didx, s_di)
            a_pe = pltpu.async_copy(pe2_hbm.at[pl.ds(base, K), :], pev, s_pe)
            a_si.wait()
            a_ga = pltpu.async_copy(qa_hbm.at[sidx], qav, s_ga)
            a_di.wait()
            a_gb = pltpu.async_copy(qb_hbm.at[didx], qbv, s_gb)
            a_pe.wait()
            a_ga.wait()
            a_gb.wait()

            @pl.loop(0, K)
            def _(r):
                z = qav[r, :] + qbv[r, :] + pev[r, :]
                ev[r, :] = _sc_tanh(z)

            pltpu.sync_copy(ev, out_hbm.at[pl.ds(base, K), :])

    return run(qa, qb, pe2, src, dst)


def kernel(x, edge_index, edge_attr, W_msg, b_msg, W_un, b_un, W_ue, b_ue):
    n, d_in = x.shape
    e, d_e = edge_attr.shape
    d_out = W_msg.shape[1]

    src = edge_index[0]
    dst = edge_index[1]

    Wm_x = W_msg[:d_in]
    Wm_e = W_msg[d_in:]
    Wu_n = W_un[:d_out]
    Wu_x = W_un[d_out:]
    Wue_s = W_ue[:d_in]
    Wue_d = W_ue[d_in : d_in + d_out]
    Wue_e = W_ue[d_in + d_out :]

    px, qa = _tc_node_pre(x, Wm_x, Wue_s, b_msg, blk=1000)
    pe, pe2 = _tc_edge_pre(edge_attr, Wm_e, Wue_e, b_ue, blk=8000)
    acc, cnt_flat = _sc_aggregate(px, pe, src, dst)
    cnts = cnt_flat.reshape(NC * NS, n).T
    h, qb = _tc_node_update(acc, cnts, x, Wu_n, Wu_x, b_un, Wue_d, blk=1000)
    e_out = _sc_edge_out(qa, qb, pe2, src, dst)
    return (h, e_out)
